# baseline (speedup 1.0000x reference)
import functools

import jax
import jax.numpy as jnp
from jax import lax
from jax.experimental import pallas as pl
from jax.experimental.pallas import tpu as pltpu
from jax.experimental.pallas import tpu_sc as plsc

N = 10000
E = 640000
NP = 10240
F = 128
BN = 1024
CHUNK = 8000
ONES_ROW = 126


def _k1_body(w_ref, x_ref, o_ref):
    y = lax.dot_general(w_ref[...], x_ref[...],
                        (((1,), (1,)), ((), ())),
                        preferred_element_type=jnp.float32)
    row = lax.broadcasted_iota(jnp.int32, y.shape, 0)
    o_ref[...] = jnp.where(row == ONES_ROW, 1.0, y)


def _project1(W1lp, x_p):
    return pl.pallas_call(
        _k1_body,
        grid=(NP // BN,),
        in_specs=[pl.BlockSpec((F, F), lambda j: (0, 0)),
                  pl.BlockSpec((BN, F), lambda j: (j, 0))],
        out_specs=pl.BlockSpec((F, BN), lambda j: (0, j)),
        out_shape=jax.ShapeDtypeStruct((F, NP), jnp.float32),
    )(W1lp, x_p)


def _k2_body(s_ref, x_ref, wr_ref, wl2_ref, sc_ref, sh_ref, h_ref, y2_ref):
    S = s_ref[...]
    invc = 1.0 / jnp.maximum(S[ONES_ROW:ONES_ROW + 1, :], 1.0)
    xr = lax.dot_general(wr_ref[...], x_ref[...],
                         (((1,), (1,)), ((), ())),
                         preferred_element_type=jnp.float32)
    h = jnp.maximum((S * invc + xr) * sc_ref[...] + sh_ref[...], 0.0)
    h_ref[...] = h
    y2_ref[...] = lax.dot_general(wl2_ref[...], h,
                                  (((1,), (0,)), ((), ())),
                                  preferred_element_type=jnp.float32)


def _layer1_epilogue(S1t, x_p, W1rp, W2lp, scale1, shift1):
    return pl.pallas_call(
        _k2_body,
        grid=(NP // BN,),
        in_specs=[pl.BlockSpec((F, BN), lambda j: (0, j)),
                  pl.BlockSpec((BN, F), lambda j: (j, 0)),
                  pl.BlockSpec((F, F), lambda j: (0, 0)),
                  pl.BlockSpec((F, F), lambda j: (0, 0)),
                  pl.BlockSpec((F, 1), lambda j: (0, 0)),
                  pl.BlockSpec((F, 1), lambda j: (0, 0))],
        out_specs=[pl.BlockSpec((F, BN), lambda j: (0, j)),
                   pl.BlockSpec((F, BN), lambda j: (0, j))],
        out_shape=[jax.ShapeDtypeStruct((F, NP), jnp.float32),
                   jax.ShapeDtypeStruct((F, NP), jnp.float32)],
    )(S1t, x_p, W1rp, W2lp, scale1, shift1)


def _k3_body(s2_ref, s1_ref, h1_ref, wr2_ref, sc_ref, sh_ref, o_ref):
    invc = 1.0 / jnp.maximum(s1_ref[ONES_ROW:ONES_ROW + 1, :], 1.0)
    xr = lax.dot_general(wr2_ref[...], h1_ref[...],
                         (((1,), (0,)), ((), ())),
                         preferred_element_type=jnp.float32)
    o_ref[...] = jnp.maximum(
        (s2_ref[...] * invc + xr) * sc_ref[...] + sh_ref[...], 0.0)


def _layer2_epilogue(S2t, S1t, h1t, W2rp, scale2, shift2):
    return pl.pallas_call(
        _k3_body,
        grid=(NP // BN,),
        in_specs=[pl.BlockSpec((F, BN), lambda j: (0, j)),
                  pl.BlockSpec((F, BN), lambda j: (0, j)),
                  pl.BlockSpec((F, BN), lambda j: (0, j)),
                  pl.BlockSpec((F, F), lambda j: (0, 0)),
                  pl.BlockSpec((F, 1), lambda j: (0, 0)),
                  pl.BlockSpec((F, 1), lambda j: (0, 0))],
        out_specs=pl.BlockSpec((F, BN), lambda j: (0, j)),
        out_shape=jax.ShapeDtypeStruct((F, NP), jnp.float32),
    )(S2t, S1t, h1t, W2rp, scale2, shift2)


_RPW = F // 32
_SLICE = _RPW * NP


_NCH = E // CHUNK


@functools.cache
def _make_sc_segsum():
    mesh = plsc.VectorSubcoreMesh(core_axis_name="c", subcore_axis_name="s",
                                  num_cores=2, num_subcores=16)
    return pl.kernel(
        _sc_segsum_body,
        mesh=mesh,
        out_type=jax.ShapeDtypeStruct((F * NP,), jnp.float32),
        scratch_types=[
            pltpu.VMEM((_SLICE,), jnp.float32),
            pltpu.VMEM((_SLICE,), jnp.float32),
            pltpu.VMEM((CHUNK,), jnp.int32),
            pltpu.VMEM((CHUNK,), jnp.int32),
            pltpu.VMEM((CHUNK,), jnp.int32),
            pltpu.VMEM((CHUNK,), jnp.int32),
            pltpu.SemaphoreType.DMA((2,)),
            pltpu.SemaphoreType.DMA((2,)),
            pltpu.SemaphoreType.DMA,
        ],
        compiler_params=pltpu.CompilerParams(needs_layout_passes=False),
    )


def _sc_segsum(yt_flat, src, dst):
    return _make_sc_segsum()(yt_flat, src, dst)


def _sc_segsum_body(yt_hbm, src_hbm, dst_hbm, out_hbm,
                    col_y, accum, src_b0, src_b1, dst_b0, dst_b1,
                    sem_s, sem_d, sem_y):
    src_b = (src_b0, src_b1)
    dst_b = (dst_b0, dst_b1)
    w = lax.axis_index("s") * 2 + lax.axis_index("c")
    base = w * _SLICE
    ycopy = pltpu.async_copy(yt_hbm.at[pl.ds(base, _SLICE)], col_y, sem_y)

    zero16 = jnp.zeros((16,), jnp.float32)

    @plsc.parallel_loop(0, _SLICE // 16, unroll=8)
    def _zero(i):
        accum[pl.ds(i * 16, 16)] = zero16

    ycopy.wait()

    def _start(c, b):
        eb = c * CHUNK
        pltpu.async_copy(src_hbm.at[pl.ds(eb, CHUNK)], src_b[b], sem_s.at[b])
        pltpu.async_copy(dst_hbm.at[pl.ds(eb, CHUNK)], dst_b[b], sem_d.at[b])

    def _wait(c, b):
        eb = c * CHUNK
        pltpu.make_async_copy(
            src_hbm.at[pl.ds(eb, CHUNK)], src_b[b], sem_s.at[b]).wait()
        pltpu.make_async_copy(
            dst_hbm.at[pl.ds(eb, CHUNK)], dst_b[b], sem_d.at[b]).wait()

    _start(0, 0)

    def _group(g, _):
        for b in range(2):
            c = g * 2 + b

            @pl.when(c + 1 < _NCH)
            def _():
                _start(c + 1, 1 - b)

            _wait(c, b)

            @plsc.parallel_loop(0, CHUNK // 16, unroll=8)
            def _edges16(i):
                s16 = src_b[b][pl.ds(i * 16, 16)]
                d16 = dst_b[b][pl.ds(i * 16, 16)]
                v = plsc.load_gather(col_y, [s16])
                plsc.addupdate_scatter(accum, [d16], v)
                for cc in range(1, _RPW):
                    off = jnp.int32(cc * NP)
                    v = plsc.load_gather(col_y, [s16 + off])
                    plsc.addupdate_scatter(accum, [d16 + off], v)

        return 0

    lax.fori_loop(0, _NCH // 2, _group, 0)
    pltpu.sync_copy(accum, out_hbm.at[pl.ds(base, _SLICE)])


def _pad_w(W):
    return jnp.zeros((F, F), jnp.float32).at[:W.shape[0], :W.shape[1]].set(W)


def _bn_fold(g, be, rm, rv, b, eps=1e-5):
    scale = g * lax.rsqrt(rv + eps)
    shift = be - rm * scale + b * scale
    scale_p = jnp.zeros((F, 1), jnp.float32).at[:scale.shape[0], 0].set(scale)
    shift_p = jnp.zeros((F, 1), jnp.float32).at[:shift.shape[0], 0].set(shift)
    return scale_p, shift_p


def kernel(x, edge_index, W1l, b1l, W1r, g1, be1, rm1, rv1,
           W2l, b2l, W2r, g2, be2, rm2, rv2):
    x_p = jnp.zeros((NP, F), jnp.float32).at[:N, :].set(x)
    src = edge_index[0]
    dst = edge_index[1]

    W1lp = _pad_w(W1l)
    W1rp = _pad_w(W1r)
    W2lp = _pad_w(W2l)
    W2rp = _pad_w(W2r)
    scale1, shift1 = _bn_fold(g1, be1, rm1, rv1, b1l)
    scale2, shift2 = _bn_fold(g2, be2, rm2, rv2, b2l)

    y1t = _project1(W1lp, x_p)
    S1t = _sc_segsum(y1t.reshape(-1), src, dst).reshape(F, NP)
    h1t, y2t = _layer1_epilogue(S1t, x_p, W1rp, W2lp, scale1, shift1)
    S2t = _sc_segsum(y2t.reshape(-1), src, dst).reshape(F, NP)
    h2t = _layer2_epilogue(S2t, S1t, h1t, W2rp, scale2, shift2)

    return h2t[:126, :N].T


# speedup vs baseline: 1.1214x; 1.1214x over previous
import functools

import jax
import jax.numpy as jnp
from jax import lax
from jax.experimental import pallas as pl
from jax.experimental.pallas import tpu as pltpu
from jax.experimental.pallas import tpu_sc as plsc

N = 10000
E = 640000
NP = 10240
F = 128
BN = 1024
CHUNK = 8000
ONES_ROW = 126


def _k1_body(w_ref, x_ref, o_ref):
    y = lax.dot_general(w_ref[...], x_ref[...],
                        (((1,), (1,)), ((), ())),
                        preferred_element_type=jnp.float32)
    row = lax.broadcasted_iota(jnp.int32, y.shape, 0)
    o_ref[...] = jnp.where(row == ONES_ROW, 1.0, y)


def _project1(W1lp, x_p):
    return pl.pallas_call(
        _k1_body,
        grid=(NP // BN,),
        in_specs=[pl.BlockSpec((F, F), lambda j: (0, 0)),
                  pl.BlockSpec((BN, F), lambda j: (j, 0))],
        out_specs=pl.BlockSpec((F, BN), lambda j: (0, j)),
        out_shape=jax.ShapeDtypeStruct((F, NP), jnp.float32),
    )(W1lp, x_p)


def _k2_body(s_ref, x_ref, wr_ref, wl2_ref, sc_ref, sh_ref, h_ref, y2_ref):
    S = s_ref[...]
    invc = 1.0 / jnp.maximum(S[ONES_ROW:ONES_ROW + 1, :], 1.0)
    xr = lax.dot_general(wr_ref[...], x_ref[...],
                         (((1,), (1,)), ((), ())),
                         preferred_element_type=jnp.float32)
    h = jnp.maximum((S * invc + xr) * sc_ref[...] + sh_ref[...], 0.0)
    h_ref[...] = h
    y2_ref[...] = lax.dot_general(wl2_ref[...], h,
                                  (((1,), (0,)), ((), ())),
                                  preferred_element_type=jnp.float32)


def _layer1_epilogue(S1t, x_p, W1rp, W2lp, scale1, shift1):
    return pl.pallas_call(
        _k2_body,
        grid=(NP // BN,),
        in_specs=[pl.BlockSpec((F, BN), lambda j: (0, j)),
                  pl.BlockSpec((BN, F), lambda j: (j, 0)),
                  pl.BlockSpec((F, F), lambda j: (0, 0)),
                  pl.BlockSpec((F, F), lambda j: (0, 0)),
                  pl.BlockSpec((F, 1), lambda j: (0, 0)),
                  pl.BlockSpec((F, 1), lambda j: (0, 0))],
        out_specs=[pl.BlockSpec((F, BN), lambda j: (0, j)),
                   pl.BlockSpec((F, BN), lambda j: (0, j))],
        out_shape=[jax.ShapeDtypeStruct((F, NP), jnp.float32),
                   jax.ShapeDtypeStruct((F, NP), jnp.float32)],
    )(S1t, x_p, W1rp, W2lp, scale1, shift1)


def _k3_body(s2_ref, s1_ref, h1_ref, wr2_ref, sc_ref, sh_ref, o_ref):
    invc = 1.0 / jnp.maximum(s1_ref[ONES_ROW:ONES_ROW + 1, :], 1.0)
    xr = lax.dot_general(wr2_ref[...], h1_ref[...],
                         (((1,), (0,)), ((), ())),
                         preferred_element_type=jnp.float32)
    o_ref[...] = jnp.maximum(
        (s2_ref[...] * invc + xr) * sc_ref[...] + sh_ref[...], 0.0)


def _layer2_epilogue(S2t, S1t, h1t, W2rp, scale2, shift2):
    return pl.pallas_call(
        _k3_body,
        grid=(NP // BN,),
        in_specs=[pl.BlockSpec((F, BN), lambda j: (0, j)),
                  pl.BlockSpec((F, BN), lambda j: (0, j)),
                  pl.BlockSpec((F, BN), lambda j: (0, j)),
                  pl.BlockSpec((F, F), lambda j: (0, 0)),
                  pl.BlockSpec((F, 1), lambda j: (0, 0)),
                  pl.BlockSpec((F, 1), lambda j: (0, 0))],
        out_specs=pl.BlockSpec((F, BN), lambda j: (0, j)),
        out_shape=jax.ShapeDtypeStruct((F, NP), jnp.float32),
    )(S2t, S1t, h1t, W2rp, scale2, shift2)


_RPW = F // 32
_SLICE = _RPW * NP


_NCH = E // CHUNK


@functools.cache
def _make_sc_segsum():
    mesh = plsc.VectorSubcoreMesh(core_axis_name="c", subcore_axis_name="s",
                                  num_cores=2, num_subcores=16)
    return pl.kernel(
        _sc_segsum_body,
        mesh=mesh,
        out_type=jax.ShapeDtypeStruct((F * NP,), jnp.float32),
        scratch_types=[
            pltpu.VMEM((_SLICE,), jnp.float32),
            pltpu.VMEM((_SLICE,), jnp.float32),
            pltpu.VMEM((CHUNK,), jnp.int32),
            pltpu.VMEM((CHUNK,), jnp.int32),
            pltpu.SemaphoreType.DMA((2,)),
            pltpu.SemaphoreType.DMA,
        ],
        compiler_params=pltpu.CompilerParams(needs_layout_passes=False),
    )


def _sc_segsum(yt_flat, comb):
    return _make_sc_segsum()(yt_flat, comb)


def _sc_segsum_body(yt_hbm, comb_hbm, out_hbm,
                    col_y, accum, eb0, eb1, sem_e, sem_y):
    edge_b = (eb0, eb1)
    w = lax.axis_index("s") * 2 + lax.axis_index("c")
    base = w * _SLICE
    ycopy = pltpu.async_copy(yt_hbm.at[pl.ds(base, _SLICE)], col_y, sem_y)

    zero16 = jnp.zeros((16,), jnp.float32)

    @plsc.parallel_loop(0, _SLICE // 16, unroll=8)
    def _zero(i):
        accum[pl.ds(i * 16, 16)] = zero16

    ycopy.wait()

    def _start(c, b):
        pltpu.async_copy(comb_hbm.at[pl.ds(c * CHUNK, CHUNK)],
                         edge_b[b], sem_e.at[b])

    def _wait(c, b):
        pltpu.make_async_copy(comb_hbm.at[pl.ds(c * CHUNK, CHUNK)],
                              edge_b[b], sem_e.at[b]).wait()

    _start(0, 0)
    mask14 = jnp.full((16,), 16383, jnp.int32)

    def _group(g, _):
        for b in range(2):
            c = g * 2 + b

            @pl.when(c + 1 < _NCH)
            def _():
                _start(c + 1, 1 - b)

            _wait(c, b)

            @plsc.parallel_loop(0, CHUNK // 16, unroll=4)
            def _edges16(i):
                cb16 = edge_b[b][pl.ds(i * 16, 16)]
                s16 = lax.shift_right_logical(cb16, 14)
                d16 = cb16 & mask14
                v = plsc.load_gather(col_y, [s16])
                plsc.addupdate_scatter(accum, [d16], v)
                for cc in range(1, _RPW):
                    off = jnp.int32(cc * NP)
                    v = plsc.load_gather(col_y, [s16 + off])
                    plsc.addupdate_scatter(accum, [d16 + off], v)

        return 0

    lax.fori_loop(0, _NCH // 2, _group, 0)
    pltpu.sync_copy(accum, out_hbm.at[pl.ds(base, _SLICE)])


def _pad_w(W):
    return jnp.zeros((F, F), jnp.float32).at[:W.shape[0], :W.shape[1]].set(W)


def _bn_fold(g, be, rm, rv, b, eps=1e-5):
    scale = g * lax.rsqrt(rv + eps)
    shift = be - rm * scale + b * scale
    scale_p = jnp.zeros((F, 1), jnp.float32).at[:scale.shape[0], 0].set(scale)
    shift_p = jnp.zeros((F, 1), jnp.float32).at[:shift.shape[0], 0].set(shift)
    return scale_p, shift_p


def kernel(x, edge_index, W1l, b1l, W1r, g1, be1, rm1, rv1,
           W2l, b2l, W2r, g2, be2, rm2, rv2):
    x_p = jnp.zeros((NP, F), jnp.float32).at[:N, :].set(x)
    comb = edge_index[0] * jnp.int32(16384) + edge_index[1]

    W1lp = _pad_w(W1l)
    W1rp = _pad_w(W1r)
    W2lp = _pad_w(W2l)
    W2rp = _pad_w(W2r)
    scale1, shift1 = _bn_fold(g1, be1, rm1, rv1, b1l)
    scale2, shift2 = _bn_fold(g2, be2, rm2, rv2, b2l)

    y1t = _project1(W1lp, x_p)
    S1t = _sc_segsum(y1t.reshape(-1), comb).reshape(F, NP)
    h1t, y2t = _layer1_epilogue(S1t, x_p, W1rp, W2lp, scale1, shift1)
    S2t = _sc_segsum(y2t.reshape(-1), comb).reshape(F, NP)
    h2t = _layer2_epilogue(S2t, S1t, h1t, W2rp, scale2, shift2)

    return h2t[:126, :N].T


# speedup vs baseline: 1.2933x; 1.1533x over previous
import functools

import jax
import jax.numpy as jnp
from jax import lax
from jax.experimental import pallas as pl
from jax.experimental.pallas import tpu as pltpu
from jax.experimental.pallas import tpu_sc as plsc

N = 10000
E = 640000
NP = 10240
F = 128
BN = 1024
CHUNK = 8000
ONES_ROW = 126


def _pack_rows(y):
    lo = lax.bitcast_convert_type(y[:F // 2].astype(jnp.bfloat16), jnp.uint16)
    hi = lax.bitcast_convert_type(y[F // 2:].astype(jnp.bfloat16), jnp.uint16)
    packed = (hi.astype(jnp.uint32) << 16) | lo.astype(jnp.uint32)
    return lax.bitcast_convert_type(packed, jnp.int32)


def _k1_body(w_ref, x_ref, o_ref):
    y = lax.dot_general(w_ref[...], x_ref[...],
                        (((1,), (1,)), ((), ())),
                        preferred_element_type=jnp.float32)
    row = lax.broadcasted_iota(jnp.int32, y.shape, 0)
    o_ref[...] = _pack_rows(jnp.where(row == ONES_ROW // 2, 1.0, y))


def _project1(W1lp_eo, x_p):
    return pl.pallas_call(
        _k1_body,
        grid=(NP // BN,),
        in_specs=[pl.BlockSpec((F, F), lambda j: (0, 0)),
                  pl.BlockSpec((BN, F), lambda j: (j, 0))],
        out_specs=pl.BlockSpec((F // 2, BN), lambda j: (0, j)),
        out_shape=jax.ShapeDtypeStruct((F // 2, NP), jnp.int32),
    )(W1lp_eo, x_p)


def _k2_body(s_ref, x_ref, wr_ref, wl2_ref, sc_ref, sh_ref, h_ref, y2_ref):
    S = s_ref[...]
    invc = 1.0 / jnp.maximum(S[ONES_ROW:ONES_ROW + 1, :], 1.0)
    xr = lax.dot_general(wr_ref[...], x_ref[...],
                         (((1,), (1,)), ((), ())),
                         preferred_element_type=jnp.float32)
    h = jnp.maximum((S * invc + xr) * sc_ref[...] + sh_ref[...], 0.0)
    h_ref[...] = h
    y2 = lax.dot_general(wl2_ref[...], h,
                         (((1,), (0,)), ((), ())),
                         preferred_element_type=jnp.float32)
    y2_ref[...] = _pack_rows(y2)


def _layer1_epilogue(S1t, x_p, W1rp, W2lp_eo, scale1, shift1):
    return pl.pallas_call(
        _k2_body,
        grid=(NP // BN,),
        in_specs=[pl.BlockSpec((F, BN), lambda j: (0, j)),
                  pl.BlockSpec((BN, F), lambda j: (j, 0)),
                  pl.BlockSpec((F, F), lambda j: (0, 0)),
                  pl.BlockSpec((F, F), lambda j: (0, 0)),
                  pl.BlockSpec((F, 1), lambda j: (0, 0)),
                  pl.BlockSpec((F, 1), lambda j: (0, 0))],
        out_specs=[pl.BlockSpec((F, BN), lambda j: (0, j)),
                   pl.BlockSpec((F // 2, BN), lambda j: (0, j))],
        out_shape=[jax.ShapeDtypeStruct((F, NP), jnp.float32),
                   jax.ShapeDtypeStruct((F // 2, NP), jnp.int32)],
    )(S1t, x_p, W1rp, W2lp_eo, scale1, shift1)


def _k3_body(s2_ref, s1_ref, h1_ref, wr2_ref, sc_ref, sh_ref, o_ref):
    invc = 1.0 / jnp.maximum(s1_ref[ONES_ROW:ONES_ROW + 1, :], 1.0)
    xr = lax.dot_general(wr2_ref[...], h1_ref[...],
                         (((1,), (0,)), ((), ())),
                         preferred_element_type=jnp.float32)
    h2 = jnp.maximum(
        (s2_ref[...] * invc + xr) * sc_ref[...] + sh_ref[...], 0.0)
    o_ref[...] = h2.T


def _layer2_epilogue(S2t, S1t, h1t, W2rp, scale2, shift2):
    return pl.pallas_call(
        _k3_body,
        grid=(NP // BN,),
        in_specs=[pl.BlockSpec((F, BN), lambda j: (0, j)),
                  pl.BlockSpec((F, BN), lambda j: (0, j)),
                  pl.BlockSpec((F, BN), lambda j: (0, j)),
                  pl.BlockSpec((F, F), lambda j: (0, 0)),
                  pl.BlockSpec((F, 1), lambda j: (0, 0)),
                  pl.BlockSpec((F, 1), lambda j: (0, 0))],
        out_specs=pl.BlockSpec((BN, F), lambda j: (j, 0)),
        out_shape=jax.ShapeDtypeStruct((NP, F), jnp.float32),
    )(S2t, S1t, h1t, W2rp, scale2, shift2)


_RPW = F // 32
_SLICE = _RPW * NP
_PPW = _RPW // 2
_PSLICE = _PPW * NP
_NCH = E // CHUNK


@functools.cache
def _make_sc_segsum():
    mesh = plsc.VectorSubcoreMesh(core_axis_name="c", subcore_axis_name="s",
                                  num_cores=2, num_subcores=16)
    return pl.kernel(
        _sc_segsum_body,
        mesh=mesh,
        out_type=jax.ShapeDtypeStruct((F * NP,), jnp.float32),
        scratch_types=[
            pltpu.VMEM((_PSLICE,), jnp.int32),
            pltpu.VMEM((_SLICE,), jnp.float32),
            pltpu.VMEM((CHUNK,), jnp.int32),
            pltpu.VMEM((CHUNK,), jnp.int32),
            pltpu.SemaphoreType.DMA((2,)),
            pltpu.SemaphoreType.DMA,
        ],
        compiler_params=pltpu.CompilerParams(needs_layout_passes=False),
    )


def _sc_segsum(yt_flat, comb):
    return _make_sc_segsum()(yt_flat, comb)


def _sc_segsum_body(yt_hbm, comb_hbm, out_hbm,
                    col_y, accum, eb0, eb1, sem_e, sem_y):
    edge_b = (eb0, eb1)
    w = lax.axis_index("s") * 2 + lax.axis_index("c")
    base = w * _SLICE
    pbase = w * _PSLICE
    ycopy = pltpu.async_copy(yt_hbm.at[pl.ds(pbase, _PSLICE)], col_y, sem_y)

    zero16 = jnp.zeros((16,), jnp.float32)

    @plsc.parallel_loop(0, _SLICE // 16, unroll=8)
    def _zero(i):
        accum[pl.ds(i * 16, 16)] = zero16

    ycopy.wait()

    def _start(c, b):
        pltpu.async_copy(comb_hbm.at[pl.ds(c * CHUNK, CHUNK)],
                         edge_b[b], sem_e.at[b])

    def _wait(c, b):
        pltpu.make_async_copy(comb_hbm.at[pl.ds(c * CHUNK, CHUNK)],
                              edge_b[b], sem_e.at[b]).wait()

    _start(0, 0)
    mask14 = jnp.full((16,), 16383, jnp.int32)

    def _group(g, _):
        for b in range(2):
            c = g * 2 + b

            @pl.when(c + 1 < _NCH)
            def _():
                _start(c + 1, 1 - b)

            _wait(c, b)

            @plsc.parallel_loop(0, CHUNK // 16, unroll=4)
            def _edges16(i):
                cb16 = edge_b[b][pl.ds(i * 16, 16)]
                s16 = lax.shift_right_logical(cb16, 14)
                d16 = cb16 & mask14
                for pp in range(_PPW):
                    sidx = s16 if pp == 0 else s16 + jnp.int32(pp * NP)
                    w16 = plsc.load_gather(col_y, [sidx])
                    bfv = plsc.bitcast(w16, jnp.bfloat16)
                    lov, hiv = plsc.unpack(
                        bfv, format=plsc.PackFormat.INTERLEAVED,
                        preferred_element_type=jnp.float32)
                    dlo = d16 if pp == 0 else d16 + jnp.int32(2 * pp * NP)
                    plsc.addupdate_scatter(accum, [dlo], lov)
                    plsc.addupdate_scatter(
                        accum, [d16 + jnp.int32((2 * pp + 1) * NP)], hiv)

        return 0

    lax.fori_loop(0, _NCH // 2, _group, 0)
    pltpu.sync_copy(accum, out_hbm.at[pl.ds(base, _SLICE)])


def _pad_w(W):
    return jnp.zeros((F, F), jnp.float32).at[:W.shape[0], :W.shape[1]].set(W)


def _bn_fold(g, be, rm, rv, b, eps=1e-5):
    scale = g * lax.rsqrt(rv + eps)
    shift = be - rm * scale + b * scale
    scale_p = jnp.zeros((F, 1), jnp.float32).at[:scale.shape[0], 0].set(scale)
    shift_p = jnp.zeros((F, 1), jnp.float32).at[:shift.shape[0], 0].set(shift)
    return scale_p, shift_p


def kernel(x, edge_index, W1l, b1l, W1r, g1, be1, rm1, rv1,
           W2l, b2l, W2r, g2, be2, rm2, rv2):
    x_p = jnp.zeros((NP, F), jnp.float32).at[:N, :].set(x)
    comb = edge_index[0] * jnp.int32(16384) + edge_index[1]

    perm_eo = jnp.concatenate([jnp.arange(0, F, 2), jnp.arange(1, F, 2)])
    W1lp_eo = _pad_w(W1l)[perm_eo]
    W1rp = _pad_w(W1r)
    W2lp_eo = _pad_w(W2l)[perm_eo]
    W2rp = _pad_w(W2r)
    scale1, shift1 = _bn_fold(g1, be1, rm1, rv1, b1l)
    scale2, shift2 = _bn_fold(g2, be2, rm2, rv2, b2l)

    y1t = _project1(W1lp_eo, x_p)
    S1t = _sc_segsum(y1t.reshape(-1), comb).reshape(F, NP)
    h1t, y2t = _layer1_epilogue(S1t, x_p, W1rp, W2lp_eo, scale1, shift1)
    S2t = _sc_segsum(y2t.reshape(-1), comb).reshape(F, NP)
    h2 = _layer2_epilogue(S2t, S1t, h1t, W2rp, scale2, shift2)

    return h2[:N, :126]
